# Initial kernel scaffold; baseline (speedup 1.0000x reference)
import jax
import jax.numpy as jnp
from jax.experimental import pallas as pl


def kernel(token_ids, embeddings):
    raise NotImplementedError("write your pallas kernel here")


# speedup vs baseline: 1.5479x; 1.5479x over previous
import functools

import jax
import jax.numpy as jnp
from jax import lax
from jax.experimental import pallas as pl
from jax.experimental.pallas import tpu as pltpu
from jax.experimental.pallas import tpu_sc as plsc

_DIM = 32
_B = 16384 * 26
_NC = 2
_NS = 16
_NW = _NC * _NS
_BPW = _B // _NW
_CHUNK = 1024
_NCHUNK = _BPW // _CHUNK

_mesh = plsc.VectorSubcoreMesh(core_axis_name="c", subcore_axis_name="s")


@functools.partial(
    pl.kernel,
    mesh=_mesh,
    out_type=jax.ShapeDtypeStruct((_B, _DIM), jnp.float32),
    scratch_types=[
        pltpu.VMEM((_CHUNK,), jnp.int32),
        pltpu.VMEM((_CHUNK, _DIM), jnp.float32),
        pltpu.SemaphoreType.DMA,
    ],
    compiler_params=pltpu.CompilerParams(use_tc_tiling_on_sc=False),
)
def _gather_kernel(idx_hbm, table_hbm, out_hbm, idx_v, rows_v, sem):
    wid = lax.axis_index("s") * _NC + lax.axis_index("c")
    base = wid * _BPW

    def body(i, carry):
        off = base + i * _CHUNK
        pltpu.sync_copy(idx_hbm.at[pl.ds(off, _CHUNK)], idx_v)
        pltpu.async_copy(table_hbm.at[idx_v], rows_v, sem).wait()
        pltpu.sync_copy(rows_v, out_hbm.at[pl.ds(off, _CHUNK)])
        return carry

    lax.fori_loop(0, _NCHUNK, body, 0)


def kernel(token_ids, embeddings):
    b, f = token_ids.shape
    idx = token_ids.reshape(-1).astype(jnp.int32)
    out = _gather_kernel(idx, embeddings)
    return out.reshape(b, f, _DIM)


# speedup vs baseline: 1.5762x; 1.0183x over previous
import functools

import jax
import jax.numpy as jnp
from jax import lax
from jax.experimental import pallas as pl
from jax.experimental.pallas import tpu as pltpu
from jax.experimental.pallas import tpu_sc as plsc

_DIM = 32
_B = 16384 * 26
_NC = 2
_NS = 16
_NW = _NC * _NS
_BPW = _B // _NW
_CHUNK = 1024
_NCHUNK = _BPW // _CHUNK
_NBUF = 3

_mesh = plsc.VectorSubcoreMesh(core_axis_name="c", subcore_axis_name="s")


@functools.partial(
    pl.kernel,
    mesh=_mesh,
    out_type=jax.ShapeDtypeStruct((_B, _DIM), jnp.float32),
    scratch_types=[
        pltpu.VMEM((_BPW,), jnp.int32),
        *[pltpu.VMEM((_CHUNK, _DIM), jnp.float32) for _ in range(_NBUF)],
        pltpu.SemaphoreType.DMA,
        *[pltpu.SemaphoreType.DMA for _ in range(_NBUF)],
        *[pltpu.SemaphoreType.DMA for _ in range(_NBUF)],
    ],
    compiler_params=pltpu.CompilerParams(use_tc_tiling_on_sc=False),
)
def _gather_kernel(idx_hbm, table_hbm, out_hbm, idx_v, *rest):
    rows = rest[:_NBUF]
    sem_i = rest[_NBUF]
    sem_g = rest[_NBUF + 1:2 * _NBUF + 1]
    sem_w = rest[2 * _NBUF + 1:]

    wid = lax.axis_index("s") * _NC + lax.axis_index("c")
    base = wid * _BPW
    pltpu.async_copy(idx_hbm.at[pl.ds(base, _BPW)], idx_v, sem_i).wait()

    gathers = [None] * _NCHUNK
    writes = [None] * _NCHUNK

    def fire_gather(i):
        b = i % _NBUF
        gathers[i] = pltpu.async_copy(
            table_hbm.at[idx_v.at[pl.ds(i * _CHUNK, _CHUNK)]],
            rows[b], sem_g[b])

    def fire_write(j):
        b = j % _NBUF
        writes[j] = pltpu.async_copy(
            rows[b], out_hbm.at[pl.ds(base + j * _CHUNK, _CHUNK)], sem_w[b])

    for i in range(_NCHUNK):
        if i >= _NBUF:
            writes[i - _NBUF].wait()
        fire_gather(i)
        if i >= _NBUF - 1:
            j = i - (_NBUF - 1)
            gathers[j].wait()
            fire_write(j)
    for j in range(_NCHUNK - (_NBUF - 1), _NCHUNK):
        gathers[j].wait()
        fire_write(j)
    for j in range(_NCHUNK - _NBUF, _NCHUNK):
        writes[j].wait()


def kernel(token_ids, embeddings):
    b, f = token_ids.shape
    idx = token_ids.reshape(-1).astype(jnp.int32)
    out = _gather_kernel(idx, embeddings)
    return out.reshape(b, f, _DIM)


# speedup vs baseline: 2.5757x; 1.6341x over previous
import functools

import jax
import jax.numpy as jnp
from jax import lax
from jax.experimental import pallas as pl
from jax.experimental.pallas import tpu as pltpu
from jax.experimental.pallas import tpu_sc as plsc

_NUM = 1000000
_DIM = 32
_BATCH = 16384
_FIELDS = 26
_B = _BATCH * _FIELDS
_NC = 2
_NS = 16
_NW = _NC * _NS
_BPW = _B // _NW
_CHUNK = 1024
_NCHUNK = _BPW // _CHUNK
_NBUF = 3

_TCOLS = 8192
_TQ = _TCOLS // 4
_TGRID = (_NUM + _TCOLS - 1) // _TCOLS


def _detranspose_body(t_ref, o_ref):
    t = t_ref[...]
    blk = jnp.concatenate(
        [t[:, q * _TQ:(q + 1) * _TQ].T for q in range(4)], axis=1)
    o_ref[...] = blk.reshape(_TQ * 128)


_detranspose = pl.pallas_call(
    _detranspose_body,
    grid=(_TGRID,),
    in_specs=[pl.BlockSpec((_DIM, _TCOLS), lambda j: (0, j))],
    out_specs=pl.BlockSpec((_TQ * 128,), lambda j: (j,)),
    out_shape=jax.ShapeDtypeStruct((_TGRID * _TQ * 128,), jnp.float32),
)

_mesh = plsc.VectorSubcoreMesh(core_axis_name="c", subcore_axis_name="s")


@functools.partial(
    pl.kernel,
    mesh=_mesh,
    out_type=jax.ShapeDtypeStruct((_B, _DIM), jnp.float32),
    scratch_types=[
        pltpu.VMEM((_BPW,), jnp.int32),
        *[pltpu.VMEM((_CHUNK, _DIM), jnp.float32) for _ in range(_NBUF)],
        pltpu.SemaphoreType.DMA,
        *[pltpu.SemaphoreType.DMA for _ in range(_NBUF)],
        *[pltpu.SemaphoreType.DMA for _ in range(_NBUF)],
    ],
    compiler_params=pltpu.CompilerParams(use_tc_tiling_on_sc=False),
)
def _gather_kernel(idx_hbm, table_hbm, out_hbm, idx_v, *rest):
    rows = rest[:_NBUF]
    sem_i = rest[_NBUF]
    sem_g = rest[_NBUF + 1:2 * _NBUF + 1]
    sem_w = rest[2 * _NBUF + 1:]

    wid = lax.axis_index("s") * _NC + lax.axis_index("c")
    base = wid * _BPW
    pltpu.async_copy(idx_hbm.at[pl.ds(base, _BPW)], idx_v, sem_i).wait()

    gathers = [None] * _NCHUNK
    writes = [None] * _NCHUNK

    def fire_gather(i):
        b = i % _NBUF
        gathers[i] = pltpu.async_copy(
            table_hbm.at[idx_v.at[pl.ds(i * _CHUNK, _CHUNK)]],
            rows[b], sem_g[b])

    def fire_write(j):
        b = j % _NBUF
        writes[j] = pltpu.async_copy(
            rows[b], out_hbm.at[pl.ds(base + j * _CHUNK, _CHUNK)], sem_w[b])

    for i in range(_NCHUNK):
        if i >= _NBUF:
            writes[i - _NBUF].wait()
        fire_gather(i)
        if i >= _NBUF - 1:
            j = i - (_NBUF - 1)
            gathers[j].wait()
            fire_write(j)
    for j in range(_NCHUNK - (_NBUF - 1), _NCHUNK):
        gathers[j].wait()
        fire_write(j)
    for j in range(_NCHUNK - _NBUF, _NCHUNK):
        writes[j].wait()


_OQ = _BATCH // 4


def _out_transpose_body(g_ref, o_ref):
    g = g_ref[...].reshape(_OQ, 128)
    blk = jnp.concatenate(
        [g[:, q * _DIM:(q + 1) * _DIM].T for q in range(4)], axis=1)
    o_ref[...] = blk.reshape(1, _DIM, _BATCH)


_out_transpose = pl.pallas_call(
    _out_transpose_body,
    grid=(_FIELDS,),
    in_specs=[pl.BlockSpec((_OQ * 128,), lambda f: (f,))],
    out_specs=pl.BlockSpec((1, _DIM, _BATCH), lambda f: (f, 0, 0)),
    out_shape=jax.ShapeDtypeStruct((_FIELDS, _DIM, _BATCH), jnp.float32),
)


def kernel(token_ids, embeddings):
    table_lin = _detranspose(embeddings.T).reshape(-1, _DIM)

    ids = token_ids.astype(jnp.int32)
    j, u = ids // _TCOLS, ids % _TCOLS
    ids = 4 * (j * _TQ + u % _TQ) + u // _TQ

    idx = ids.T.reshape(_FIELDS, 4, _OQ).swapaxes(1, 2).reshape(-1)

    g = _gather_kernel(idx, table_lin)
    o2 = _out_transpose(g.reshape(-1))
    return o2.transpose(2, 0, 1)
